# baseline (speedup 1.0000x reference)
import functools

import jax
import jax.numpy as jnp
import numpy as np
from jax import lax
from jax.experimental import pallas as pl
from jax.experimental.pallas import tpu as pltpu
from jax.experimental.pallas import tpu_sc as plsc

_N_BINS = 20
_NB_PAD = 32
_BOUNDS = [float(b) for b in np.linspace(0.0, 1.0, _N_BINS + 1).astype(np.float32)]

_NW = 32
_SC_CH = 128
_SC_W_ROWS = 6400
_N_SC = _NW * _SC_W_ROWS
_TC_ROWS = 1600
_TC_DEPTH = 8


def _make_bounds_col():
    b = np.full((_NB_PAD, 128), np.inf, dtype=np.float32)
    b[: _N_BINS + 1, 0] = np.linspace(0.0, 1.0, _N_BINS + 1).astype(np.float32)
    return b


def _tc_body(x_hbm, lab_hbm, b_ref, out_ref, xbuf, lbuf, xsem, lsem):
    n, c = x_hbm.shape
    chunk0 = _N_SC // _TC_ROWS
    nchunk = n // _TC_ROWS - chunk0

    def xcopy(i, slot):
        return pltpu.make_async_copy(
            x_hbm.at[pl.ds((chunk0 + i) * _TC_ROWS, _TC_ROWS), :], xbuf.at[slot],
            xsem.at[slot])

    def lcopy(i, slot):
        return pltpu.make_async_copy(
            lab_hbm.at[chunk0 + i], lbuf.at[slot], lsem.at[slot])

    for d in range(min(_TC_DEPTH, nchunk)):
        xcopy(d, d).start()
        lcopy(d, d).start()

    bounds = b_ref[...][:, 0:1]

    def loop(i, part):
        slot = jax.lax.rem(i, _TC_DEPTH)
        xcopy(i, slot).wait()
        lcopy(i, slot).wait()
        x = xbuf[slot]
        lab = lbuf[slot]
        xt = x.T
        conf = jnp.max(xt, axis=0, keepdims=True)
        pred = jnp.argmax(xt, axis=0).astype(jnp.int32)[None, :]
        acc = (pred == lab).astype(jnp.float32)

        m = (conf > bounds).astype(jnp.float32)
        vt = jnp.concatenate([jnp.ones_like(conf), conf, acc], axis=0)
        upd = jax.lax.dot_general(
            vt, m, (((1,), (1,)), ((), ())), preferred_element_type=jnp.float32)

        @pl.when(i + _TC_DEPTH < nchunk)
        def _():
            xcopy(i + _TC_DEPTH, slot).start()
            lcopy(i + _TC_DEPTH, slot).start()

        return part + upd

    part = jax.lax.fori_loop(
        0, nchunk, loop, jnp.zeros((3, _NB_PAD), jnp.float32))
    out_ref[...] = jnp.concatenate(
        [part, jnp.zeros((5, _NB_PAD), jnp.float32)], axis=0)


@jax.jit
def _tc_partial(softmaxes, labels):
    n, c = softmaxes.shape
    nchunk = n // _TC_ROWS
    lab3 = labels.reshape(nchunk, 1, _TC_ROWS)
    bounds_col = jnp.asarray(_make_bounds_col())
    out = pl.pallas_call(
        _tc_body,
        in_specs=[
            pl.BlockSpec(memory_space=pltpu.HBM),
            pl.BlockSpec(memory_space=pltpu.HBM),
            pl.BlockSpec(memory_space=pltpu.VMEM),
        ],
        out_specs=pl.BlockSpec(memory_space=pltpu.VMEM),
        out_shape=jax.ShapeDtypeStruct((8, _NB_PAD), jnp.float32),
        scratch_shapes=[
            pltpu.VMEM((_TC_DEPTH, _TC_ROWS, 100), jnp.float32),
            pltpu.VMEM((_TC_DEPTH, 1, _TC_ROWS), jnp.int32),
            pltpu.SemaphoreType.DMA((_TC_DEPTH,)),
            pltpu.SemaphoreType.DMA((_TC_DEPTH,)),
        ],
    )(softmaxes, lab3, bounds_col)
    return out[:3, :]


def _sc_body(x_hbm, lab_hbm, out_hbm, xbuf, lbuf, accum, xsem, lsem):
    nc = 2
    wid = lax.axis_index("s") * nc + lax.axis_index("c")
    row0 = wid * _SC_W_ROWS
    nchunk = _SC_W_ROWS // _SC_CH

    def xcopy(k, slot):
        return pltpu.make_async_copy(
            x_hbm.at[pl.ds(row0 + k * _SC_CH, _SC_CH), :], xbuf.at[slot],
            xsem.at[slot])

    def lcopy(k, slot):
        return pltpu.make_async_copy(
            lab_hbm.at[pl.ds(row0 + k * _SC_CH, _SC_CH)], lbuf.at[slot],
            lsem.at[slot])

    zero16 = jnp.zeros((16,), jnp.float32)
    for r in range(3):
        accum[r, pl.ds(0, 16)] = zero16
        accum[r, pl.ds(16, 16)] = zero16

    for d in range(2):
        xcopy(d, d).start()
        lcopy(d, d).start()

    lanes = lax.iota(jnp.int32, 16)

    def process_chunk(k, slot):
        def group_loop(g, carry2):
            rows = g * 16 + lanes

            def class_loop(cc, mx_arg):
                mx, arg = mx_arg
                v = plsc.load_gather(
                    xbuf.at[slot], [rows, jnp.full((16,), cc, jnp.int32)])
                better = v > mx
                mx = jnp.where(better, v, mx)
                arg = jnp.where(better, jnp.full((16,), cc, jnp.int32), arg)
                return (mx, arg)

            mx0 = jnp.full((16,), -jnp.inf, jnp.float32)
            arg0 = jnp.zeros((16,), jnp.int32)
            conf, pred = lax.fori_loop(0, 100, class_loop, (mx0, arg0))

            lab = lbuf[slot, pl.ds(g * 16, 16)]
            accv = jnp.where(pred == lab, 1.0, 0.0).astype(jnp.float32)

            idx = jnp.zeros((16,), jnp.int32)
            for b in _BOUNDS:
                idx = idx + jnp.where(conf > b, 1, 0).astype(jnp.int32)
            valid = (idx > 0) & (idx <= _N_BINS)
            bidx = jnp.where(valid, idx - 1, 0)
            ones = jnp.ones((16,), jnp.float32)
            plsc.addupdate_scatter(accum.at[0], [bidx], ones, mask=valid)
            plsc.addupdate_scatter(accum.at[1], [bidx], conf, mask=valid)
            plsc.addupdate_scatter(accum.at[2], [bidx], accv, mask=valid)
            return carry2

        lax.fori_loop(0, _SC_CH // 16, group_loop, 0)

    def pair_loop(j, carry):
        for b in range(2):
            k = 2 * j + b
            xcopy(k, b).wait()
            lcopy(k, b).wait()
            process_chunk(k, b)

            @pl.when(k + 2 < nchunk)
            def _():
                xcopy(k + 2, b).start()
                lcopy(k + 2, b).start()

        return carry

    lax.fori_loop(0, nchunk // 2, pair_loop, 0)

    pltpu.sync_copy(accum, out_hbm.at[wid])


@jax.jit
def _sc_partial(softmaxes, labels):
    mesh = plsc.VectorSubcoreMesh(core_axis_name="c", subcore_axis_name="s")
    kern = pl.kernel(
        _sc_body,
        out_type=jax.ShapeDtypeStruct((_NW, 3, _NB_PAD), jnp.float32),
        mesh=mesh,
        compiler_params=pltpu.CompilerParams(needs_layout_passes=False),
        scratch_types=[
            pltpu.VMEM((2, _SC_CH, 100), jnp.float32),
            pltpu.VMEM((2, _SC_CH), jnp.int32),
            pltpu.VMEM((3, _NB_PAD), jnp.float32),
            pltpu.SemaphoreType.DMA((2,)),
            pltpu.SemaphoreType.DMA((2,)),
        ],
    )
    return kern(softmaxes, labels)


def kernel(softmaxes, labels):
    n = softmaxes.shape[0]
    sc_part = _sc_partial(softmaxes, labels)
    tc_part = _tc_partial(softmaxes, labels)

    cum = tc_part
    per_bin_tc = (cum - jnp.concatenate([cum[:, 1:], cum[:, -1:]], axis=1))[:, :_N_BINS]
    per_bin_sc = jnp.sum(sc_part, axis=0)[:, :_N_BINS]
    cnt, sum_conf, sum_acc = tuple(per_bin_tc + per_bin_sc)

    denom = jnp.maximum(cnt, 1.0)
    nonzero = cnt > 0.0
    acc_bin = jnp.where(nonzero, sum_acc / denom, 0.0)
    conf_bin = jnp.where(nonzero, sum_conf / denom, 0.0)
    prop = cnt / float(n)
    ece = jnp.sum(jnp.where(nonzero, jnp.abs(conf_bin - acc_bin) * prop, 0.0),
                  keepdims=True)
    return ece, acc_bin


# speedup vs baseline: 1.2768x; 1.2768x over previous
import functools

import jax
import jax.numpy as jnp
import numpy as np
from jax import lax
from jax.experimental import pallas as pl
from jax.experimental.pallas import tpu as pltpu
from jax.experimental.pallas import tpu_sc as plsc

_N_BINS = 20
_NB_PAD = 32
_BOUNDS = [float(b) for b in np.linspace(0.0, 1.0, _N_BINS + 1).astype(np.float32)]

_NW = 32
_SC_CH = 128
_SC_W_ROWS = 6400
_N_SC = 0
_TC_ROWS = 1600
_TC_DEPTH = 8


def _make_bounds_col():
    b = np.full((_NB_PAD, 128), np.inf, dtype=np.float32)
    b[: _N_BINS + 1, 0] = np.linspace(0.0, 1.0, _N_BINS + 1).astype(np.float32)
    return b


def _tc_body(x_hbm, lab_hbm, b_ref, out_ref, xbuf, lbuf, xsem, lsem):
    n, c = x_hbm.shape
    chunk0 = _N_SC // _TC_ROWS
    nchunk = n // _TC_ROWS - chunk0

    def xcopy(i, slot):
        return pltpu.make_async_copy(
            x_hbm.at[pl.ds((chunk0 + i) * _TC_ROWS, _TC_ROWS), :], xbuf.at[slot],
            xsem.at[slot])

    def lcopy(i, slot):
        return pltpu.make_async_copy(
            lab_hbm.at[chunk0 + i], lbuf.at[slot], lsem.at[slot])

    for d in range(min(_TC_DEPTH, nchunk)):
        xcopy(d, d).start()
        lcopy(d, d).start()

    bounds = b_ref[...][:, 0:1]

    def loop(i, part):
        slot = jax.lax.rem(i, _TC_DEPTH)
        xcopy(i, slot).wait()
        lcopy(i, slot).wait()
        x = xbuf[slot]
        lab = lbuf[slot]
        xt = x.T
        conf = jnp.max(xt, axis=0, keepdims=True)
        pred = jnp.argmax(xt, axis=0).astype(jnp.int32)[None, :]
        acc = (pred == lab).astype(jnp.float32)

        m = (conf > bounds).astype(jnp.float32)
        vt = jnp.concatenate([jnp.ones_like(conf), conf, acc], axis=0)
        upd = jax.lax.dot_general(
            vt, m, (((1,), (1,)), ((), ())), preferred_element_type=jnp.float32)

        @pl.when(i + _TC_DEPTH < nchunk)
        def _():
            xcopy(i + _TC_DEPTH, slot).start()
            lcopy(i + _TC_DEPTH, slot).start()

        return part + upd

    part = jax.lax.fori_loop(
        0, nchunk, loop, jnp.zeros((3, _NB_PAD), jnp.float32))
    out_ref[...] = jnp.concatenate(
        [part, jnp.zeros((5, _NB_PAD), jnp.float32)], axis=0)


@jax.jit
def _tc_partial(softmaxes, labels):
    n, c = softmaxes.shape
    nchunk = n // _TC_ROWS
    lab3 = labels.reshape(nchunk, 1, _TC_ROWS)
    bounds_col = jnp.asarray(_make_bounds_col())
    out = pl.pallas_call(
        _tc_body,
        in_specs=[
            pl.BlockSpec(memory_space=pltpu.HBM),
            pl.BlockSpec(memory_space=pltpu.HBM),
            pl.BlockSpec(memory_space=pltpu.VMEM),
        ],
        out_specs=pl.BlockSpec(memory_space=pltpu.VMEM),
        out_shape=jax.ShapeDtypeStruct((8, _NB_PAD), jnp.float32),
        scratch_shapes=[
            pltpu.VMEM((_TC_DEPTH, _TC_ROWS, 100), jnp.float32),
            pltpu.VMEM((_TC_DEPTH, 1, _TC_ROWS), jnp.int32),
            pltpu.SemaphoreType.DMA((_TC_DEPTH,)),
            pltpu.SemaphoreType.DMA((_TC_DEPTH,)),
        ],
    )(softmaxes, lab3, bounds_col)
    return out[:3, :]


def _sc_body(x_hbm, lab_hbm, out_hbm, xbuf, lbuf, accum, xsem, lsem):
    nc = 2
    wid = lax.axis_index("s") * nc + lax.axis_index("c")
    row0 = wid * _SC_W_ROWS
    nchunk = _SC_W_ROWS // _SC_CH

    def xcopy(k, slot):
        return pltpu.make_async_copy(
            x_hbm.at[pl.ds(row0 + k * _SC_CH, _SC_CH), :], xbuf.at[slot],
            xsem.at[slot])

    def lcopy(k, slot):
        return pltpu.make_async_copy(
            lab_hbm.at[pl.ds(row0 + k * _SC_CH, _SC_CH)], lbuf.at[slot],
            lsem.at[slot])

    zero16 = jnp.zeros((16,), jnp.float32)
    for r in range(3):
        accum[r, pl.ds(0, 16)] = zero16
        accum[r, pl.ds(16, 16)] = zero16

    for d in range(2):
        xcopy(d, d).start()
        lcopy(d, d).start()

    lanes = lax.iota(jnp.int32, 16)

    def process_chunk(k, slot):
        def group_loop(g, carry2):
            rows = g * 16 + lanes

            def class_loop(cc, mx_arg):
                mx, arg = mx_arg
                v = plsc.load_gather(
                    xbuf.at[slot], [rows, jnp.full((16,), cc, jnp.int32)])
                better = v > mx
                mx = jnp.where(better, v, mx)
                arg = jnp.where(better, jnp.full((16,), cc, jnp.int32), arg)
                return (mx, arg)

            mx0 = jnp.full((16,), -jnp.inf, jnp.float32)
            arg0 = jnp.zeros((16,), jnp.int32)
            conf, pred = lax.fori_loop(0, 100, class_loop, (mx0, arg0))

            lab = lbuf[slot, pl.ds(g * 16, 16)]
            accv = jnp.where(pred == lab, 1.0, 0.0).astype(jnp.float32)

            idx = jnp.zeros((16,), jnp.int32)
            for b in _BOUNDS:
                idx = idx + jnp.where(conf > b, 1, 0).astype(jnp.int32)
            valid = (idx > 0) & (idx <= _N_BINS)
            bidx = jnp.where(valid, idx - 1, 0)
            ones = jnp.ones((16,), jnp.float32)
            plsc.addupdate_scatter(accum.at[0], [bidx], ones, mask=valid)
            plsc.addupdate_scatter(accum.at[1], [bidx], conf, mask=valid)
            plsc.addupdate_scatter(accum.at[2], [bidx], accv, mask=valid)
            return carry2

        lax.fori_loop(0, _SC_CH // 16, group_loop, 0)

    def pair_loop(j, carry):
        for b in range(2):
            k = 2 * j + b
            xcopy(k, b).wait()
            lcopy(k, b).wait()
            process_chunk(k, b)

            @pl.when(k + 2 < nchunk)
            def _():
                xcopy(k + 2, b).start()
                lcopy(k + 2, b).start()

        return carry

    lax.fori_loop(0, nchunk // 2, pair_loop, 0)

    pltpu.sync_copy(accum, out_hbm.at[wid])


@jax.jit
def _sc_partial(softmaxes, labels):
    mesh = plsc.VectorSubcoreMesh(core_axis_name="c", subcore_axis_name="s")
    kern = pl.kernel(
        _sc_body,
        out_type=jax.ShapeDtypeStruct((_NW, 3, _NB_PAD), jnp.float32),
        mesh=mesh,
        compiler_params=pltpu.CompilerParams(needs_layout_passes=False),
        scratch_types=[
            pltpu.VMEM((2, _SC_CH, 100), jnp.float32),
            pltpu.VMEM((2, _SC_CH), jnp.int32),
            pltpu.VMEM((3, _NB_PAD), jnp.float32),
            pltpu.SemaphoreType.DMA((2,)),
            pltpu.SemaphoreType.DMA((2,)),
        ],
    )
    return kern(softmaxes, labels)


def kernel(softmaxes, labels):
    n = softmaxes.shape[0]
    tc_part = _tc_partial(softmaxes, labels)

    cum = tc_part
    per_bin_tc = (cum - jnp.concatenate([cum[:, 1:], cum[:, -1:]], axis=1))[:, :_N_BINS]
    cnt, sum_conf, sum_acc = tuple(per_bin_tc)

    denom = jnp.maximum(cnt, 1.0)
    nonzero = cnt > 0.0
    acc_bin = jnp.where(nonzero, sum_acc / denom, 0.0)
    conf_bin = jnp.where(nonzero, sum_conf / denom, 0.0)
    prop = cnt / float(n)
    ece = jnp.sum(jnp.where(nonzero, jnp.abs(conf_bin - acc_bin) * prop, 0.0),
                  keepdims=True)
    return ece, acc_bin


# speedup vs baseline: 1.5500x; 1.2139x over previous
import functools

import jax
import jax.numpy as jnp
import numpy as np
from jax import lax
from jax.experimental import pallas as pl
from jax.experimental.pallas import tpu as pltpu
from jax.experimental.pallas import tpu_sc as plsc

_N_BINS = 20
_NB_PAD = 32
_BOUNDS = [float(b) for b in np.linspace(0.0, 1.0, _N_BINS + 1).astype(np.float32)]

_NW = 32
_SC_CH = 128
_SC_W_ROWS = 6400
_N_SC = 0
_TC_ROWS = 8000
_TC_DEPTH = 4


def _make_bounds_col():
    b = np.full((_NB_PAD, 128), np.inf, dtype=np.float32)
    b[: _N_BINS + 1, 0] = np.linspace(0.0, 1.0, _N_BINS + 1).astype(np.float32)
    return b


def _tc_body(x_hbm, lab_hbm, b_ref, out_ref, xbuf, lbuf, xsem, lsem):
    n, c = x_hbm.shape
    chunk0 = _N_SC // _TC_ROWS
    nchunk = n // _TC_ROWS - chunk0

    def xcopy(i, slot):
        return pltpu.make_async_copy(
            x_hbm.at[pl.ds((chunk0 + i) * _TC_ROWS, _TC_ROWS), :], xbuf.at[slot],
            xsem.at[slot])

    def lcopy(i, slot):
        return pltpu.make_async_copy(
            lab_hbm.at[chunk0 + i], lbuf.at[slot], lsem.at[slot])

    for d in range(min(_TC_DEPTH, nchunk)):
        xcopy(d, d).start()
        lcopy(d, d).start()

    bounds = b_ref[...][:, 0:1]

    def loop(i, part):
        slot = jax.lax.rem(i, _TC_DEPTH)
        xcopy(i, slot).wait()
        lcopy(i, slot).wait()
        x = xbuf[slot]
        lab = lbuf[slot]
        xt = x.T
        conf = jnp.max(xt, axis=0, keepdims=True)
        pred = jnp.argmax(xt, axis=0).astype(jnp.int32)[None, :]
        acc = (pred == lab).astype(jnp.float32)

        m = (conf > bounds).astype(jnp.float32)
        vt = jnp.concatenate([jnp.ones_like(conf), conf, acc], axis=0)
        upd = jax.lax.dot_general(
            vt, m, (((1,), (1,)), ((), ())), preferred_element_type=jnp.float32)

        @pl.when(i + _TC_DEPTH < nchunk)
        def _():
            xcopy(i + _TC_DEPTH, slot).start()
            lcopy(i + _TC_DEPTH, slot).start()

        return part + upd

    part = jax.lax.fori_loop(
        0, nchunk, loop, jnp.zeros((3, _NB_PAD), jnp.float32))
    out_ref[...] = jnp.concatenate(
        [part, jnp.zeros((5, _NB_PAD), jnp.float32)], axis=0)


@jax.jit
def _tc_partial(softmaxes, labels):
    n, c = softmaxes.shape
    nchunk = n // _TC_ROWS
    lab3 = labels.reshape(nchunk, 1, _TC_ROWS)
    bounds_col = jnp.asarray(_make_bounds_col())
    out = pl.pallas_call(
        _tc_body,
        in_specs=[
            pl.BlockSpec(memory_space=pltpu.HBM),
            pl.BlockSpec(memory_space=pltpu.HBM),
            pl.BlockSpec(memory_space=pltpu.VMEM),
        ],
        out_specs=pl.BlockSpec(memory_space=pltpu.VMEM),
        out_shape=jax.ShapeDtypeStruct((8, _NB_PAD), jnp.float32),
        scratch_shapes=[
            pltpu.VMEM((_TC_DEPTH, _TC_ROWS, 100), jnp.float32),
            pltpu.VMEM((_TC_DEPTH, 1, _TC_ROWS), jnp.int32),
            pltpu.SemaphoreType.DMA((_TC_DEPTH,)),
            pltpu.SemaphoreType.DMA((_TC_DEPTH,)),
        ],
    )(softmaxes, lab3, bounds_col)
    return out[:3, :]


def _sc_body(x_hbm, lab_hbm, out_hbm, xbuf, lbuf, accum, xsem, lsem):
    nc = 2
    wid = lax.axis_index("s") * nc + lax.axis_index("c")
    row0 = wid * _SC_W_ROWS
    nchunk = _SC_W_ROWS // _SC_CH

    def xcopy(k, slot):
        return pltpu.make_async_copy(
            x_hbm.at[pl.ds(row0 + k * _SC_CH, _SC_CH), :], xbuf.at[slot],
            xsem.at[slot])

    def lcopy(k, slot):
        return pltpu.make_async_copy(
            lab_hbm.at[pl.ds(row0 + k * _SC_CH, _SC_CH)], lbuf.at[slot],
            lsem.at[slot])

    zero16 = jnp.zeros((16,), jnp.float32)
    for r in range(3):
        accum[r, pl.ds(0, 16)] = zero16
        accum[r, pl.ds(16, 16)] = zero16

    for d in range(2):
        xcopy(d, d).start()
        lcopy(d, d).start()

    lanes = lax.iota(jnp.int32, 16)

    def process_chunk(k, slot):
        def group_loop(g, carry2):
            rows = g * 16 + lanes

            def class_loop(cc, mx_arg):
                mx, arg = mx_arg
                v = plsc.load_gather(
                    xbuf.at[slot], [rows, jnp.full((16,), cc, jnp.int32)])
                better = v > mx
                mx = jnp.where(better, v, mx)
                arg = jnp.where(better, jnp.full((16,), cc, jnp.int32), arg)
                return (mx, arg)

            mx0 = jnp.full((16,), -jnp.inf, jnp.float32)
            arg0 = jnp.zeros((16,), jnp.int32)
            conf, pred = lax.fori_loop(0, 100, class_loop, (mx0, arg0))

            lab = lbuf[slot, pl.ds(g * 16, 16)]
            accv = jnp.where(pred == lab, 1.0, 0.0).astype(jnp.float32)

            idx = jnp.zeros((16,), jnp.int32)
            for b in _BOUNDS:
                idx = idx + jnp.where(conf > b, 1, 0).astype(jnp.int32)
            valid = (idx > 0) & (idx <= _N_BINS)
            bidx = jnp.where(valid, idx - 1, 0)
            ones = jnp.ones((16,), jnp.float32)
            plsc.addupdate_scatter(accum.at[0], [bidx], ones, mask=valid)
            plsc.addupdate_scatter(accum.at[1], [bidx], conf, mask=valid)
            plsc.addupdate_scatter(accum.at[2], [bidx], accv, mask=valid)
            return carry2

        lax.fori_loop(0, _SC_CH // 16, group_loop, 0)

    def pair_loop(j, carry):
        for b in range(2):
            k = 2 * j + b
            xcopy(k, b).wait()
            lcopy(k, b).wait()
            process_chunk(k, b)

            @pl.when(k + 2 < nchunk)
            def _():
                xcopy(k + 2, b).start()
                lcopy(k + 2, b).start()

        return carry

    lax.fori_loop(0, nchunk // 2, pair_loop, 0)

    pltpu.sync_copy(accum, out_hbm.at[wid])


@jax.jit
def _sc_partial(softmaxes, labels):
    mesh = plsc.VectorSubcoreMesh(core_axis_name="c", subcore_axis_name="s")
    kern = pl.kernel(
        _sc_body,
        out_type=jax.ShapeDtypeStruct((_NW, 3, _NB_PAD), jnp.float32),
        mesh=mesh,
        compiler_params=pltpu.CompilerParams(needs_layout_passes=False),
        scratch_types=[
            pltpu.VMEM((2, _SC_CH, 100), jnp.float32),
            pltpu.VMEM((2, _SC_CH), jnp.int32),
            pltpu.VMEM((3, _NB_PAD), jnp.float32),
            pltpu.SemaphoreType.DMA((2,)),
            pltpu.SemaphoreType.DMA((2,)),
        ],
    )
    return kern(softmaxes, labels)


def kernel(softmaxes, labels):
    n = softmaxes.shape[0]
    tc_part = _tc_partial(softmaxes, labels)

    cum = tc_part
    per_bin_tc = (cum - jnp.concatenate([cum[:, 1:], cum[:, -1:]], axis=1))[:, :_N_BINS]
    cnt, sum_conf, sum_acc = tuple(per_bin_tc)

    denom = jnp.maximum(cnt, 1.0)
    nonzero = cnt > 0.0
    acc_bin = jnp.where(nonzero, sum_acc / denom, 0.0)
    conf_bin = jnp.where(nonzero, sum_conf / denom, 0.0)
    prop = cnt / float(n)
    ece = jnp.sum(jnp.where(nonzero, jnp.abs(conf_bin - acc_bin) * prop, 0.0),
                  keepdims=True)
    return ece, acc_bin
